# baseline (speedup 1.0000x reference)
import functools

import jax
import jax.numpy as jnp
from jax import lax
from jax.experimental import pallas as pl
from jax.experimental.pallas import tpu as pltpu
from jax.experimental.pallas import tpu_sc as plsc

B = 16384
FC = 13
NF = 26
NE = 32
NOUT = FC + NF
NC = 2
NS = 16
NW = NC * NS
BLK = 1024
SUB = 128
NSUB = BLK // SUB
CB = 128
BPW = B // NW

_mesh = plsc.VectorSubcoreMesh(
    core_axis_name="c", subcore_axis_name="s", num_cores=NC, num_subcores=NS
)


@functools.partial(
    pl.kernel,
    out_type=jax.ShapeDtypeStruct((B, NOUT, NE), jnp.float32),
    mesh=_mesh,
    compiler_params=pltpu.CompilerParams(use_tc_tiling_on_sc=False),
    scratch_types=[
        pltpu.VMEM((NSUB, SUB), jnp.int32),
        pltpu.VMEM((BLK, NE), jnp.float32),
        pltpu.VMEM((NF, NE), jnp.float32),
        pltpu.VMEM((NF, 16), jnp.int32),
        pltpu.VMEM((FC, NE), jnp.float32),
        pltpu.VMEM((FC, NE), jnp.float32),
        pltpu.VMEM((16,), jnp.float32),
        pltpu.VMEM((CB, 16), jnp.float32),
        pltpu.VMEM((CB, FC, NE), jnp.float32),
        pltpu.SemaphoreType.DMA,
    ],
)
def _sc_embed(xc_hbm, xcat_t_hbm, wnan_hbm, offs_hbm, wcat_hbm, bcat_hbm,
              wcont_hbm, bcont_hbm, out_hbm,
              idx2, rows, biasv, offsv, wcv, bcv, wnanv, xcv, cstage, sem):
    c = lax.axis_index("c")
    s = lax.axis_index("s")
    wid = s * NC + c

    pltpu.sync_copy(bcat_hbm, biasv)
    pltpu.sync_copy(offs_hbm, offsv)
    pltpu.sync_copy(wcont_hbm, wcv)
    pltpu.sync_copy(bcont_hbm, bcv)
    pltpu.sync_copy(wnan_hbm, wnanv)

    wnv = wnanv[pl.ds(0, 16)]
    for cb in range(BPW // CB):
        b0 = wid * BPW + cb * CB
        pltpu.sync_copy(xc_hbm.at[pl.ds(b0, CB)], xcv)

        def cbody(r, _, wnv=wnv):
            xrow = xcv[r, pl.ds(0, 16)]
            for f in range(FC):
                sv = xrow[f]
                sv = jnp.where(sv != sv, wnv[f], sv)
                xb = jnp.full((16,), sv, dtype=jnp.float32)
                wv0 = wcv[f, pl.ds(0, 16)]
                wv1 = wcv[f, pl.ds(16, 16)]
                bv0 = bcv[f, pl.ds(0, 16)]
                bv1 = bcv[f, pl.ds(16, 16)]
                cstage[r, f, pl.ds(0, 16)] = xb * wv0 + bv0
                cstage[r, f, pl.ds(16, 16)] = xb * wv1 + bv1
            return 0

        lax.fori_loop(0, CB, cbody, 0, unroll=2)
        pltpu.sync_copy(cstage, out_hbm.at[pl.ds(b0, CB), pl.ds(0, FC)])

    blk = wid % 16
    grp = wid // 16
    b0 = blk * BLK
    for k in range(FC):
        f = grp * FC + k
        off_vec = offsv[f, pl.ds(0, 16)]

        for j in range(NSUB):
            pltpu.sync_copy(xcat_t_hbm.at[f, pl.ds(b0 + j * SUB, SUB)],
                            idx2.at[j])

        def obody(j, _, off_vec=off_vec):
            for i in range(SUB // 16):
                idx2[j, pl.ds(i * 16, 16)] += off_vec
            return 0

        lax.fori_loop(0, NSUB, obody, 0)

        copies = [
            pltpu.async_copy(wcat_hbm.at[idx2.at[j]],
                             rows.at[pl.ds(j * SUB, SUB)], sem)
            for j in range(NSUB)
        ]
        for cp in copies:
            cp.wait()

        bv0 = biasv[f, pl.ds(0, 16)]
        bv1 = biasv[f, pl.ds(16, 16)]

        def bbody(r, _, bv0=bv0, bv1=bv1):
            rows[r, pl.ds(0, 16)] += bv0
            rows[r, pl.ds(16, 16)] += bv1
            return 0

        lax.fori_loop(0, BLK, bbody, 0, unroll=8)

        pltpu.sync_copy(rows, out_hbm.at[pl.ds(b0, BLK), FC + f])


def kernel(x_continuous, x_categorical, w_nan, offsets, w_categorical,
           b_categorical, w_continuous, b_continuous):
    xcat_t = x_categorical.T
    xc_pad = jnp.pad(x_continuous, ((0, 0), (0, 16 - FC)))
    wnan_pad = jnp.pad(w_nan, (0, 16 - FC))
    offs2 = jnp.broadcast_to(offsets[:, None], (NF, 16))
    return _sc_embed(xc_pad, xcat_t, wnan_pad, offs2, w_categorical,
                     b_categorical, w_continuous, b_continuous)
